# Initial kernel scaffold; baseline (speedup 1.0000x reference)
import jax
import jax.numpy as jnp
from jax import lax
from jax.experimental import pallas as pl
from jax.experimental.pallas import tpu as pltpu
from jax.experimental.pallas import tpu_sc as plsc

V = 1000000
D = 64
B = 16384
K = 20

NC = 2
NS = 16
L = 16
NW = NC * NS

ITEMS_PER_W = B // NW
CHUNK = 128
NCHUNK = ITEMS_PER_W // CHUNK
UNROLL = 8


def _sc_scores_body(w_in, w_out, center, context, neg_t, out,
                    idx_c, idx_x, idx_n, rows_c, rows_x, rows_n, sc_v,
                    sem_c, sem_x, sem_n0, sem_n1):
    w = lax.axis_index("s") * NC + lax.axis_index("c")
    sems = (sem_n0, sem_n1)

    def dot_groups(rows_other, negate):
        def group(g, _):
            row = lax.iota(jnp.int32, L) + g * L

            def dstep(t, acc):
                for u in range(UNROLL):
                    col = jnp.full((L,), t * UNROLL + u, jnp.int32)
                    a = plsc.load_gather(rows_c, [row, col])
                    b = plsc.load_gather(rows_other, [row, col])
                    acc = acc + a * b
                return acc

            acc = lax.fori_loop(0, D // UNROLL, dstep,
                                jnp.zeros((L,), jnp.float32))
            sc_v[pl.ds(g * L, L)] = -acc if negate else acc
            return 0

        lax.fori_loop(0, CHUNK // L, group, 0)

    def chunk_body(c, _):
        base = w * ITEMS_PER_W + c * CHUNK
        pltpu.sync_copy(center.at[pl.ds(base, CHUNK)], idx_c)
        pltpu.sync_copy(context.at[pl.ds(base, CHUNK)], idx_x)
        cp_c = pltpu.async_copy(w_in.at[idx_c], rows_c, sem_c)
        cp_x = pltpu.async_copy(w_out.at[idx_x], rows_x, sem_x)
        pltpu.sync_copy(neg_t.at[0, pl.ds(base, CHUNK)], idx_n.at[0])
        handles = {0: pltpu.async_copy(w_out.at[idx_n.at[0]], rows_n.at[0],
                                       sems[0])}
        cp_c.wait()
        cp_x.wait()
        dot_groups(rows_x, negate=False)
        pltpu.sync_copy(sc_v, out.at[0, pl.ds(base, CHUNK)])
        for k in range(K):
            cur = k % 2
            nxt = 1 - cur
            if k + 1 < K:
                pltpu.sync_copy(neg_t.at[k + 1, pl.ds(base, CHUNK)],
                                idx_n.at[nxt])
                handles[k + 1] = pltpu.async_copy(
                    w_out.at[idx_n.at[nxt]], rows_n.at[nxt], sems[nxt])
            handles[k].wait()
            dot_groups(rows_n.at[cur], negate=True)
            pltpu.sync_copy(sc_v, out.at[k + 1, pl.ds(base, CHUNK)])
        return 0

    lax.fori_loop(0, NCHUNK, chunk_body, 0)


def _sc_scores(w_in, w_out, center, context, neg_t):
    mesh = plsc.VectorSubcoreMesh(core_axis_name="c", subcore_axis_name="s",
                                  num_cores=NC, num_subcores=NS)
    fn = pl.kernel(
        _sc_scores_body,
        out_type=jax.ShapeDtypeStruct((K + 1, B), jnp.float32),
        mesh=mesh,
        scratch_types=[
            pltpu.VMEM((CHUNK,), jnp.int32),
            pltpu.VMEM((CHUNK,), jnp.int32),
            pltpu.VMEM((2, CHUNK), jnp.int32),
            pltpu.VMEM((CHUNK, D), jnp.float32),
            pltpu.VMEM((CHUNK, D), jnp.float32),
            pltpu.VMEM((2, CHUNK, D), jnp.float32),
            pltpu.VMEM((CHUNK,), jnp.float32),
            pltpu.SemaphoreType.DMA,
            pltpu.SemaphoreType.DMA,
            pltpu.SemaphoreType.DMA,
            pltpu.SemaphoreType.DMA,
        ],
    )
    return fn(w_in, w_out, center, context, neg_t)


def _loss_body(s_ref, o_ref):
    x = s_ref[...]
    losses = -jnp.log(jax.nn.sigmoid(x) + 1e-10)
    o_ref[0, 0] = jnp.sum(losses) * (1.0 / B)


def _loss(scores):
    out = pl.pallas_call(
        _loss_body,
        out_shape=jax.ShapeDtypeStruct((1, 1), jnp.float32),
    )(scores)
    return out[0, 0]


def kernel(center, context, negatives, W_in, W_out):
    center = center.astype(jnp.int32)
    context = context.astype(jnp.int32)
    neg_t = negatives.astype(jnp.int32).T
    scores = _sc_scores(W_in, W_out, center, context, neg_t)
    return _loss(scores)


# speedup vs baseline: 3.4590x; 3.4590x over previous
import jax
import jax.numpy as jnp
from jax import lax
from jax.experimental import pallas as pl
from jax.experimental.pallas import tpu as pltpu
from jax.experimental.pallas import tpu_sc as plsc

V = 1000000
D = 64
B = 16384
K = 20

NC = 2
NS = 16
L = 16
NW = NC * NS

ITEMS_PER_W = B // NW
CHUNK = 128
NCHUNK = ITEMS_PER_W // CHUNK
UNROLL = 8


def _sc_scores_body(w_in, w_out, center, context, neg_t, out,
                    idx_c, idx_x, idx_n0, idx_n1,
                    rows_c, rows_x, rows_n0, rows_n1, sc_v,
                    sem_c, sem_x, sem_n0, sem_n1):
    w = lax.axis_index("s") * NC + lax.axis_index("c")
    idx_n = (idx_n0, idx_n1)
    rows_n = (rows_n0, rows_n1)
    sems = (sem_n0, sem_n1)

    def dot_groups(rows_other, negate):
        def group(g, _):
            row = lax.iota(jnp.int32, L) + g * L

            def dstep(t, acc):
                for u in range(UNROLL):
                    col = jnp.full((L,), t * UNROLL + u, jnp.int32)
                    a = plsc.load_gather(rows_c, [row, col])
                    b = plsc.load_gather(rows_other, [row, col])
                    acc = acc + a * b
                return acc

            acc = lax.fori_loop(0, D // UNROLL, dstep,
                                jnp.zeros((L,), jnp.float32))
            sc_v[pl.ds(g * L, L)] = -acc if negate else acc
            return 0

        lax.fori_loop(0, CHUNK // L, group, 0)

    def chunk_body(c, _):
        base = w * ITEMS_PER_W + c * CHUNK
        pltpu.sync_copy(center.at[pl.ds(base, CHUNK)], idx_c)
        pltpu.sync_copy(context.at[pl.ds(base, CHUNK)], idx_x)
        cp_c = pltpu.async_copy(w_in.at[idx_c], rows_c, sem_c)
        cp_x = pltpu.async_copy(w_out.at[idx_x], rows_x, sem_x)
        pltpu.sync_copy(neg_t.at[pl.ds(base, CHUNK)], idx_n[0])
        handles = {0: pltpu.async_copy(w_out.at[idx_n[0]], rows_n[0],
                                       sems[0])}
        cp_c.wait()
        cp_x.wait()
        dot_groups(rows_x, negate=False)
        pltpu.sync_copy(sc_v, out.at[pl.ds(base, CHUNK)])
        for k in range(K):
            cur = k % 2
            nxt = 1 - cur
            if k + 1 < K:
                pltpu.sync_copy(neg_t.at[pl.ds((k + 1) * B + base, CHUNK)],
                                idx_n[nxt])
                handles[k + 1] = pltpu.async_copy(
                    w_out.at[idx_n[nxt]], rows_n[nxt], sems[nxt])
            handles[k].wait()
            dot_groups(rows_n[cur], negate=True)
            pltpu.sync_copy(sc_v, out.at[pl.ds((k + 1) * B + base, CHUNK)])
        return 0

    lax.fori_loop(0, NCHUNK, chunk_body, 0)


def _sc_scores(w_in, w_out, center, context, neg_t):
    mesh = plsc.VectorSubcoreMesh(core_axis_name="c", subcore_axis_name="s",
                                  num_cores=NC, num_subcores=NS)
    fn = pl.kernel(
        _sc_scores_body,
        out_type=jax.ShapeDtypeStruct(((K + 1) * B,), jnp.float32),
        mesh=mesh,
        compiler_params=pltpu.CompilerParams(
            needs_layout_passes=False, use_tc_tiling_on_sc=False),
        scratch_types=[
            pltpu.VMEM((CHUNK,), jnp.int32),
            pltpu.VMEM((CHUNK,), jnp.int32),
            pltpu.VMEM((CHUNK,), jnp.int32),
            pltpu.VMEM((CHUNK,), jnp.int32),
            pltpu.VMEM((CHUNK, D), jnp.float32),
            pltpu.VMEM((CHUNK, D), jnp.float32),
            pltpu.VMEM((CHUNK, D), jnp.float32),
            pltpu.VMEM((CHUNK, D), jnp.float32),
            pltpu.VMEM((CHUNK,), jnp.float32),
            pltpu.SemaphoreType.DMA,
            pltpu.SemaphoreType.DMA,
            pltpu.SemaphoreType.DMA,
            pltpu.SemaphoreType.DMA,
        ],
    )
    return fn(w_in, w_out, center, context, neg_t)


def _loss_body(s_ref, o_ref):
    x = s_ref[...]
    losses = -jnp.log(jax.nn.sigmoid(x) + 1e-10)
    o_ref[...] = jnp.reshape(jnp.sum(losses) * (1.0 / B), (1, 1))


def _loss(scores2d):
    out = pl.pallas_call(
        _loss_body,
        out_shape=jax.ShapeDtypeStruct((1, 1), jnp.float32),
    )(scores2d)
    return out[0, 0]


def kernel(center, context, negatives, W_in, W_out):
    center = center.astype(jnp.int32)
    context = context.astype(jnp.int32)
    neg_t = negatives.astype(jnp.int32).T.reshape(K * B)
    scores = _sc_scores(W_in, W_out, center, context, neg_t)
    return _loss(scores.reshape(K + 1, B))


# speedup vs baseline: 3.5197x; 1.0175x over previous
import jax
import jax.numpy as jnp
from jax import lax
from jax.experimental import pallas as pl
from jax.experimental.pallas import tpu as pltpu
from jax.experimental.pallas import tpu_sc as plsc

V = 1000000
D = 64
B = 16384
K = 20

NC = 2
NS = 16
L = 16
NW = NC * NS

ITEMS_PER_W = B // NW
CHUNK = 128
NCHUNK = ITEMS_PER_W // CHUNK
UNROLL = 8


def _sc_scores_body(w_in, w_out, center, context, neg_flat, out,
                    idx_c, idx_x, idx_n0, idx_n1, negs_v,
                    rows_c, rows_x, rows_n0, rows_n1, sc_v,
                    sem_c, sem_x, sem_n0, sem_n1):
    w = lax.axis_index("s") * NC + lax.axis_index("c")
    idx_n = (idx_n0, idx_n1)
    rows_n = (rows_n0, rows_n1)
    sems = (sem_n0, sem_n1)

    def extract_k(k, dst):
        def j_body(j, _):
            lanes = lax.iota(jnp.int32, L) * K + (j * (L * K) + k)
            dst[pl.ds(j * L, L)] = plsc.load_gather(negs_v, [lanes])
            return 0

        lax.fori_loop(0, CHUNK // L, j_body, 0)

    def dot_groups(rows_other, negate):
        def group(g, _):
            row = lax.iota(jnp.int32, L) + g * L

            def dstep(t, acc):
                for u in range(UNROLL):
                    col = jnp.full((L,), t * UNROLL + u, jnp.int32)
                    a = plsc.load_gather(rows_c, [row, col])
                    b = plsc.load_gather(rows_other, [row, col])
                    acc = acc + a * b
                return acc

            acc = lax.fori_loop(0, D // UNROLL, dstep,
                                jnp.zeros((L,), jnp.float32))
            sc_v[pl.ds(g * L, L)] = -acc if negate else acc
            return 0

        lax.fori_loop(0, CHUNK // L, group, 0)

    def chunk_body(c, _):
        base = w * ITEMS_PER_W + c * CHUNK
        pltpu.sync_copy(center.at[pl.ds(base, CHUNK)], idx_c)
        pltpu.sync_copy(context.at[pl.ds(base, CHUNK)], idx_x)
        cp_c = pltpu.async_copy(w_in.at[idx_c], rows_c, sem_c)
        cp_x = pltpu.async_copy(w_out.at[idx_x], rows_x, sem_x)
        pltpu.sync_copy(neg_flat.at[pl.ds(base * K, CHUNK * K)], negs_v)
        extract_k(0, idx_n[0])
        handles = {0: pltpu.async_copy(w_out.at[idx_n[0]], rows_n[0],
                                       sems[0])}
        cp_c.wait()
        cp_x.wait()
        dot_groups(rows_x, negate=False)
        pltpu.sync_copy(sc_v, out.at[pl.ds(base, CHUNK)])
        for k in range(K):
            cur = k % 2
            nxt = 1 - cur
            if k + 1 < K:
                extract_k(k + 1, idx_n[nxt])
                handles[k + 1] = pltpu.async_copy(
                    w_out.at[idx_n[nxt]], rows_n[nxt], sems[nxt])
            handles[k].wait()
            dot_groups(rows_n[cur], negate=True)
            pltpu.sync_copy(sc_v, out.at[pl.ds((k + 1) * B + base, CHUNK)])
        return 0

    lax.fori_loop(0, NCHUNK, chunk_body, 0)


def _sc_scores(w_in, w_out, center, context, neg_flat):
    mesh = plsc.VectorSubcoreMesh(core_axis_name="c", subcore_axis_name="s",
                                  num_cores=NC, num_subcores=NS)
    fn = pl.kernel(
        _sc_scores_body,
        out_type=jax.ShapeDtypeStruct(((K + 1) * B,), jnp.float32),
        mesh=mesh,
        compiler_params=pltpu.CompilerParams(
            needs_layout_passes=False, use_tc_tiling_on_sc=False),
        scratch_types=[
            pltpu.VMEM((CHUNK,), jnp.int32),
            pltpu.VMEM((CHUNK,), jnp.int32),
            pltpu.VMEM((CHUNK,), jnp.int32),
            pltpu.VMEM((CHUNK,), jnp.int32),
            pltpu.VMEM((CHUNK * K,), jnp.int32),
            pltpu.VMEM((CHUNK, D), jnp.float32),
            pltpu.VMEM((CHUNK, D), jnp.float32),
            pltpu.VMEM((CHUNK, D), jnp.float32),
            pltpu.VMEM((CHUNK, D), jnp.float32),
            pltpu.VMEM((CHUNK,), jnp.float32),
            pltpu.SemaphoreType.DMA,
            pltpu.SemaphoreType.DMA,
            pltpu.SemaphoreType.DMA,
            pltpu.SemaphoreType.DMA,
        ],
    )
    return fn(w_in, w_out, center, context, neg_flat)


def _loss_body(s_ref, o_ref):
    x = s_ref[...]
    losses = -jnp.log(jax.nn.sigmoid(x) + 1e-10)
    o_ref[...] = jnp.reshape(jnp.sum(losses) * (1.0 / B), (1, 1))


def _loss(scores2d):
    out = pl.pallas_call(
        _loss_body,
        out_shape=jax.ShapeDtypeStruct((1, 1), jnp.float32),
    )(scores2d)
    return out[0, 0]


def kernel(center, context, negatives, W_in, W_out):
    center = center.astype(jnp.int32)
    context = context.astype(jnp.int32)
    neg_flat = negatives.astype(jnp.int32).reshape(B * K)
    scores = _sc_scores(W_in, W_out, center, context, neg_flat)
    return _loss(scores.reshape((K + 1) * B // 128, 128))


# speedup vs baseline: 4.2477x; 1.2068x over previous
import jax
import jax.numpy as jnp
from jax import lax
from jax.experimental import pallas as pl
from jax.experimental.pallas import tpu as pltpu
from jax.experimental.pallas import tpu_sc as plsc

V = 1000000
D = 64
B = 16384
K = 20

NC = 2
NS = 16
L = 16
NW = NC * NS

ITEMS_PER_W = B // NW
CHUNK = 64
NCHUNK = ITEMS_PER_W // CHUNK
KRES = 10
NROUND = K // KRES
UNROLL = 4
BLK = (K + 1) * CHUNK


def _sc_scores_body(w_in, w_out, center, context, neg_flat, out, *refs):
    (idx_c, idx_x, negs_v, idx_all, rows_c, rows_x, sc_all) = refs[:7]
    nbuf = (refs[7:7 + KRES], refs[7 + KRES:7 + 2 * KRES])
    sem_c, sem_x, sem_r0, sem_r1 = refs[7 + 2 * KRES:]
    sem_r = (sem_r0, sem_r1)
    w = lax.axis_index("s") * NC + lax.axis_index("c")

    def extract_indices():
        for k in range(K):
            def j_body(j, _, k=k):
                lanes = lax.iota(jnp.int32, L) * K + (j * (L * K) + k)
                idx_all[pl.ds(k * CHUNK + j * L, L)] = (
                    plsc.load_gather(negs_v, [lanes]))
                return 0

            lax.fori_loop(0, CHUNK // L, j_body, 0)

    def pos_compute():
        def group(g, _):
            row = lax.iota(jnp.int32, L) + g * L

            def dstep(t, accs):
                a0, a1 = accs
                for u in range(UNROLL):
                    dd = t * (2 * UNROLL) + 2 * u
                    c0 = jnp.full((L,), dd, jnp.int32)
                    c1 = jnp.full((L,), dd + 1, jnp.int32)
                    a0 = a0 + (plsc.load_gather(rows_c, [row, c0])
                               * plsc.load_gather(rows_x, [row, c0]))
                    a1 = a1 + (plsc.load_gather(rows_c, [row, c1])
                               * plsc.load_gather(rows_x, [row, c1]))
                return (a0, a1)

            z = jnp.zeros((L,), jnp.float32)
            a0, a1 = lax.fori_loop(0, D // (2 * UNROLL), dstep, (z, z))
            sc_all[pl.ds(g * L, L)] = a0 + a1
            return 0

        lax.fori_loop(0, CHUNK // L, group, 0)

    def round_compute(r, koff):
        bufs = nbuf[r]

        def group(g, _):
            row = lax.iota(jnp.int32, L) + g * L

            def dstep(t, accs):
                for u in range(UNROLL):
                    col = jnp.full((L,), t * UNROLL + u, jnp.int32)
                    cvec = plsc.load_gather(rows_c, [row, col])
                    accs = tuple(
                        accs[kk] + cvec * plsc.load_gather(bufs[kk],
                                                           [row, col])
                        for kk in range(KRES))
                return accs

            z = jnp.zeros((L,), jnp.float32)
            accs = lax.fori_loop(0, D // UNROLL, dstep, (z,) * KRES)
            for kk in range(KRES):
                sc_all[pl.ds((1 + koff + kk) * CHUNK + g * L, L)] = -accs[kk]
            return 0

        lax.fori_loop(0, CHUNK // L, group, 0)

    def issue_round(r, koff):
        handles = []
        for kk in range(KRES):
            handles.append(pltpu.async_copy(
                w_out.at[idx_all.at[pl.ds((koff + kk) * CHUNK, CHUNK)]],
                nbuf[r][kk], sem_r[r]))
        return handles

    def chunk_body(c, _):
        base = w * ITEMS_PER_W + c * CHUNK
        pltpu.sync_copy(center.at[pl.ds(base, CHUNK)], idx_c)
        pltpu.sync_copy(context.at[pl.ds(base, CHUNK)], idx_x)
        cp_c = pltpu.async_copy(w_in.at[idx_c], rows_c, sem_c)
        cp_x = pltpu.async_copy(w_out.at[idx_x], rows_x, sem_x)
        pltpu.sync_copy(neg_flat.at[pl.ds(base * K, CHUNK * K)], negs_v)
        extract_indices()
        h0 = issue_round(0, 0)
        cp_c.wait()
        cp_x.wait()
        pos_compute()
        h1 = issue_round(1, KRES)
        for h in h0:
            h.wait()
        round_compute(0, 0)
        for h in h1:
            h.wait()
        round_compute(1, KRES)
        pltpu.sync_copy(sc_all,
                        out.at[pl.ds((w * NCHUNK + c) * BLK, BLK)])
        return 0

    lax.fori_loop(0, NCHUNK, chunk_body, 0)


def _sc_scores(w_in, w_out, center, context, neg_flat):
    mesh = plsc.VectorSubcoreMesh(core_axis_name="c", subcore_axis_name="s",
                                  num_cores=NC, num_subcores=NS)
    scratch = [
        pltpu.VMEM((CHUNK,), jnp.int32),
        pltpu.VMEM((CHUNK,), jnp.int32),
        pltpu.VMEM((CHUNK * K,), jnp.int32),
        pltpu.VMEM((CHUNK * K,), jnp.int32),
        pltpu.VMEM((CHUNK, D), jnp.float32),
        pltpu.VMEM((CHUNK, D), jnp.float32),
        pltpu.VMEM((BLK,), jnp.float32),
    ]
    scratch += [pltpu.VMEM((CHUNK, D), jnp.float32)
                for _ in range(2 * KRES)]
    scratch += [pltpu.SemaphoreType.DMA] * 4
    fn = pl.kernel(
        _sc_scores_body,
        out_type=jax.ShapeDtypeStruct((NW * ITEMS_PER_W // CHUNK * BLK,),
                                      jnp.float32),
        mesh=mesh,
        compiler_params=pltpu.CompilerParams(
            needs_layout_passes=False, use_tc_tiling_on_sc=False),
        scratch_types=scratch,
    )
    return fn(w_in, w_out, center, context, neg_flat)


def _loss_body(s_ref, o_ref):
    x = s_ref[...]
    losses = -jnp.log(jax.nn.sigmoid(x) + 1e-10)
    o_ref[...] = jnp.reshape(jnp.sum(losses) * (1.0 / B), (1, 1))


def _loss(scores2d):
    out = pl.pallas_call(
        _loss_body,
        out_shape=jax.ShapeDtypeStruct((1, 1), jnp.float32),
    )(scores2d)
    return out[0, 0]


def kernel(center, context, negatives, W_in, W_out):
    center = center.astype(jnp.int32)
    context = context.astype(jnp.int32)
    neg_flat = negatives.astype(jnp.int32).reshape(B * K)
    scores = _sc_scores(W_in, W_out, center, context, neg_flat)
    return _loss(scores.reshape((K + 1) * B // 128, 128))
